# baseline (speedup 1.0000x reference)
import functools

import jax
import jax.numpy as jnp
from jax import lax
from jax.experimental import pallas as pl
from jax.experimental.pallas import tpu as pltpu
from jax.experimental.pallas import tpu_sc as plsc

VOCAB = 1000000
B = 16384
H = 4
D = 64
NC = 2
NS = 16
NW = NC * NS
S_PER_W = B // NW
C = 128
G = S_PER_W // C


def _sc_body(ids_hbm, table_hbm, out_hbm, idx_v, rows_v, pooled_v, sem):
    wid = lax.axis_index("s") * NC + lax.axis_index("c")

    for g in range(G):
        base = wid * S_PER_W + g * C
        for h in range(H):
            pltpu.sync_copy(ids_hbm.at[pl.ds(h * B + base, C)], idx_v.at[h])
        copies = [
            pltpu.make_async_copy(table_hbm.at[idx_v.at[h]], rows_v.at[h], sem)
            for h in range(H)
        ]
        for cp in copies:
            cp.start()
        for cp in copies:
            cp.wait()

        def pool_row(s, _):
            for d in range(D // 16):
                sl = pl.ds(d * 16, 16)
                acc = rows_v[0, s, sl]
                acc = acc + rows_v[1, s, sl]
                acc = acc + rows_v[2, s, sl]
                acc = acc + rows_v[3, s, sl]
                pooled_v[s, sl] = acc
            return _

        lax.fori_loop(0, C, pool_row, 0, unroll=2)

        pltpu.sync_copy(pooled_v, out_hbm.at[pl.ds(base, C)])


_sc_gather_pool = functools.partial(
    pl.kernel,
    out_type=jax.ShapeDtypeStruct((B, D), jnp.float32),
    mesh=plsc.VectorSubcoreMesh(core_axis_name="c", subcore_axis_name="s"),
    scratch_types=[
        pltpu.VMEM((H, C), jnp.int32),
        pltpu.VMEM((H, C, D), jnp.float32),
        pltpu.VMEM((C, D), jnp.float32),
        pltpu.SemaphoreType.DMA,
    ],
    compiler_params=pltpu.CompilerParams(use_tc_tiling_on_sc=False),
)(_sc_body)


def _proj_body(xt_ref, w_ref, o_ref):
    o_ref[...] = jax.lax.dot_general(
        xt_ref[...], w_ref[...], (((0,), (0,)), ((), ())),
        preferred_element_type=jnp.float32,
    )


def _tc_project(table_t, w):
    blk = 16384
    n = (VOCAB + blk - 1) // blk
    return pl.pallas_call(
        _proj_body,
        grid=(n,),
        in_specs=[
            pl.BlockSpec((D, blk), lambda i: (0, i)),
            pl.BlockSpec((D, D), lambda i: (0, 0)),
        ],
        out_specs=pl.BlockSpec((blk, D), lambda i: (i, 0)),
        out_shape=jax.ShapeDtypeStruct((VOCAB, D), jnp.float32),
    )(table_t, w)


def _act_body(x_ref, b_ref, o_ref):
    o_ref[...] = jnp.maximum(x_ref[...] + b_ref[...], 0.0).T


def _tc_bias_relu(x, b):
    blk = 2048
    return pl.pallas_call(
        _act_body,
        grid=(B // blk,),
        in_specs=[
            pl.BlockSpec((blk, D), lambda i: (i, 0)),
            pl.BlockSpec((1, D), lambda i: (0, 0)),
        ],
        out_specs=pl.BlockSpec((D, blk), lambda i: (0, i)),
        out_shape=jax.ShapeDtypeStruct((D, B), jnp.float32),
    )(x, b)


def kernel(ids, emb_table, proj_w, proj_b):
    ids_t = ids.T.reshape(-1)
    wt = proj_w.T * (1.0 / H)
    tw = _tc_project(emb_table.T, wt)
    pooled = _sc_gather_pool(ids_t, tw)
    out_t = _tc_bias_relu(pooled, proj_b.reshape(1, D))
    return out_t.T


# speedup vs baseline: 3.0721x; 3.0721x over previous
import functools

import jax
import jax.numpy as jnp
from jax import lax
from jax.experimental import pallas as pl
from jax.experimental.pallas import tpu as pltpu
from jax.experimental.pallas import tpu_sc as plsc

VOCAB = 1000000
B = 16384
H = 4
D = 64
NC = 2
NS = 16
NW = NC * NS
S_PER_W = B // NW
C = 128
G = S_PER_W // C
BP = 8192
BLKS = (VOCAB + 2 * BP - 1) // (2 * BP)
PAIRS = BLKS * BP
VROWS = 2 * PAIRS
_LASTBLK = VOCAB // BP


def _sc_body(idx_hbm, table_hbm, out_hbm, idx_v, rows_v, pooled_v, sem):
    wid = lax.axis_index("s") * NC + lax.axis_index("c")

    for g in range(G):
        base = wid * S_PER_W + g * C
        for h in range(H):
            pltpu.sync_copy(idx_hbm.at[pl.ds(h * B + base, C)], idx_v.at[h])
        copies = [
            pltpu.make_async_copy(table_hbm.at[idx_v.at[h]], rows_v.at[h], sem)
            for h in range(H)
        ]
        for cp in copies:
            cp.start()
        for cp in copies:
            cp.wait()

        def pool_row(s, _):
            for d in range(D // 16):
                sl = pl.ds(d * 16, 16)
                acc = rows_v[0, s, sl]
                acc = acc + rows_v[1, s, sl]
                acc = acc + rows_v[2, s, sl]
                acc = acc + rows_v[3, s, sl]
                pooled_v[s, sl] = acc
            return _

        lax.fori_loop(0, C, pool_row, 0, unroll=2)

        pltpu.sync_copy(pooled_v, out_hbm.at[pl.ds(base, C)])


_sc_gather_pool = functools.partial(
    pl.kernel,
    out_type=jax.ShapeDtypeStruct((B, D), jnp.float32),
    mesh=plsc.VectorSubcoreMesh(core_axis_name="c", subcore_axis_name="s"),
    scratch_types=[
        pltpu.VMEM((H, C), jnp.int32),
        pltpu.VMEM((H, C, D), jnp.float32),
        pltpu.VMEM((C, D), jnp.float32),
        pltpu.SemaphoreType.DMA,
    ],
    compiler_params=pltpu.CompilerParams(use_tc_tiling_on_sc=False),
)(_sc_body)


def _proj_body(xa_ref, xb_ref, w2_ref, o_ref):
    x2 = jnp.concatenate([xa_ref[...], xb_ref[...]], axis=0)
    o_ref[...] = jax.lax.dot_general(
        x2, w2_ref[...], (((0,), (0,)), ((), ())),
        preferred_element_type=jnp.float32,
    )


def _tc_project(table_t, w2):
    return pl.pallas_call(
        _proj_body,
        grid=(BLKS,),
        in_specs=[
            pl.BlockSpec((D, BP), lambda i: (0, jnp.minimum(2 * i, _LASTBLK))),
            pl.BlockSpec(
                (D, BP), lambda i: (0, jnp.minimum(2 * i + 1, _LASTBLK))),
            pl.BlockSpec((2 * D, 2 * D), lambda i: (0, 0)),
        ],
        out_specs=pl.BlockSpec((BP, 2 * D), lambda i: (i, 0)),
        out_shape=jax.ShapeDtypeStruct((PAIRS, 2 * D), jnp.float32),
    )(table_t, table_t, w2)


def _act_body(x_ref, b_ref, o_ref):
    o_ref[...] = jnp.maximum(x_ref[...] + b_ref[...], 0.0).T


def _tc_bias_relu(x, b):
    blk = 2048
    return pl.pallas_call(
        _act_body,
        grid=(B // blk,),
        in_specs=[
            pl.BlockSpec((blk, D), lambda i: (i, 0)),
            pl.BlockSpec((1, D), lambda i: (0, 0)),
        ],
        out_specs=pl.BlockSpec((D, blk), lambda i: (0, i)),
        out_shape=jax.ShapeDtypeStruct((D, B), jnp.float32),
    )(x, b)


def kernel(ids, emb_table, proj_w, proj_b):
    ids_t = ids.T.reshape(-1)
    blk = jax.lax.shift_right_logical(ids_t, 14)
    j = jnp.bitwise_and(ids_t, 2 * BP - 1)
    idx2 = jax.lax.shift_left(blk, 14) + jnp.where(
        j < BP,
        jax.lax.shift_left(j, 1),
        jax.lax.shift_left(j - BP, 1) + 1,
    )
    wt = proj_w.T * (1.0 / H)
    w2 = jnp.zeros((2 * D, 2 * D), jnp.float32)
    w2 = w2.at[:D, :D].set(wt).at[D:, D:].set(wt)
    tw = _tc_project(emb_table.T, w2)
    tw_rows = tw.reshape(VROWS, D)
    pooled = _sc_gather_pool(idx2, tw_rows)
    out_t = _tc_bias_relu(pooled, proj_b.reshape(1, D))
    return out_t.T


# speedup vs baseline: 3.1040x; 1.0104x over previous
import functools

import jax
import jax.numpy as jnp
from jax import lax
from jax.experimental import pallas as pl
from jax.experimental.pallas import tpu as pltpu
from jax.experimental.pallas import tpu_sc as plsc

VOCAB = 1000000
B = 16384
H = 4
D = 64
NC = 2
NS = 16
NW = NC * NS
S_PER_W = B // NW
C = 128
G = S_PER_W // C
BP = 8192
BLKS = (VOCAB + 2 * BP - 1) // (2 * BP)
PAIRS = BLKS * BP
VROWS = 2 * PAIRS
_LASTBLK = VOCAB // BP


def _sc_body(idx_hbm, table_hbm, bias_hbm, out_hbm, idx_v, rows_v,
             pooled_v, bias_v, sems):
    wid = lax.axis_index("s") * NC + lax.axis_index("c")
    pltpu.sync_copy(bias_hbm, bias_v)

    def load_chunk(g, buf):
        base = wid * S_PER_W + g * C
        for h in range(H):
            pltpu.sync_copy(idx_hbm.at[pl.ds(h * B + base, C)],
                            idx_v.at[buf, h])
        for h in range(H):
            for v in range(C // 16):
                sl = pl.ds(v * 16, 16)
                x = idx_v[buf, h, sl]
                blk = jax.lax.shift_right_logical(x, 14)
                j = jnp.bitwise_and(x, 2 * BP - 1)
                row = jax.lax.shift_left(blk, 14) + jnp.where(
                    j < BP,
                    jax.lax.shift_left(j, 1),
                    jax.lax.shift_left(j - BP, 1) + 1,
                )
                idx_v[buf, h, sl] = row
        return [
            pltpu.make_async_copy(table_hbm.at[idx_v.at[buf, h]],
                                  rows_v.at[buf, h], sems.at[buf])
            for h in range(H)
        ]

    def pool_chunk(g, buf, copies):
        for cp in copies:
            cp.wait()

        def pool_row(s, _):
            for d in range(D // 16):
                sl = pl.ds(d * 16, 16)
                acc = rows_v[buf, 0, s, sl]
                acc = acc + rows_v[buf, 1, s, sl]
                acc = acc + rows_v[buf, 2, s, sl]
                acc = acc + rows_v[buf, 3, s, sl]
                pooled_v[s, sl] = jnp.maximum(acc + bias_v[sl], 0.0)
            return _

        lax.fori_loop(0, C, pool_row, 0, unroll=2)
        base = wid * S_PER_W + g * C
        pltpu.sync_copy(pooled_v, out_hbm.at[pl.ds(base, C)])

    copies = load_chunk(0, 0)
    for cp in copies:
        cp.start()
    for g in range(1, G):
        nxt = load_chunk(g, g % 2)
        for cp in nxt:
            cp.start()
        pool_chunk(g - 1, (g - 1) % 2, copies)
        copies = nxt
    pool_chunk(G - 1, (G - 1) % 2, copies)


_sc_gather_pool = functools.partial(
    pl.kernel,
    out_type=jax.ShapeDtypeStruct((B, D), jnp.float32),
    mesh=plsc.VectorSubcoreMesh(core_axis_name="c", subcore_axis_name="s"),
    scratch_types=[
        pltpu.VMEM((2, H, C), jnp.int32),
        pltpu.VMEM((2, H, C, D), jnp.float32),
        pltpu.VMEM((C, D), jnp.float32),
        pltpu.VMEM((D,), jnp.float32),
        pltpu.SemaphoreType.DMA((2,)),
    ],
    compiler_params=pltpu.CompilerParams(use_tc_tiling_on_sc=False),
)(_sc_body)


def _proj_body(xa_ref, xb_ref, w2_ref, o_ref):
    x2 = jnp.concatenate([xa_ref[...], xb_ref[...]], axis=0)
    o_ref[...] = jax.lax.dot_general(
        x2, w2_ref[...], (((0,), (0,)), ((), ())),
        preferred_element_type=jnp.float32,
    )


def _tc_project(table_t, w2):
    return pl.pallas_call(
        _proj_body,
        grid=(BLKS,),
        in_specs=[
            pl.BlockSpec((D, BP), lambda i: (0, jnp.minimum(2 * i, _LASTBLK))),
            pl.BlockSpec(
                (D, BP), lambda i: (0, jnp.minimum(2 * i + 1, _LASTBLK))),
            pl.BlockSpec((2 * D, 2 * D), lambda i: (0, 0)),
        ],
        out_specs=pl.BlockSpec((BP, 2 * D), lambda i: (i, 0)),
        out_shape=jax.ShapeDtypeStruct((PAIRS, 2 * D), jnp.float32),
    )(table_t, table_t, w2)


def kernel(ids, emb_table, proj_w, proj_b):
    ids_t = ids.T.reshape(-1)
    wt = proj_w.T * (1.0 / H)
    w2 = jnp.zeros((2 * D, 2 * D), jnp.float32)
    w2 = w2.at[:D, :D].set(wt).at[D:, D:].set(wt)
    tw = _tc_project(emb_table.T, w2)
    tw_rows = tw.reshape(VROWS, D)
    return _sc_gather_pool(ids_t, tw_rows, proj_b)
